# baseline (speedup 1.0000x reference)
import jax
import jax.numpy as jnp
from jax.experimental import pallas as pl
from jax.experimental.pallas import tpu as pltpu

_COMMITMENT_COST = 0.25
_M = 256
_W = 2048


def _vq_kernel(z_ref, cb_ref, zq_ref, ids_ref, loss_ref, cnorm_ref):
    i = pl.program_id(0)
    cb = cb_ref[...]
    k = cb.shape[0]
    cbh = cb.astype(jnp.bfloat16)

    @pl.when(i == 0)
    def _init():
        cnorm_ref[...] = jnp.sum(cb * cb, axis=1).reshape(1, -1)
        loss_ref[...] = jnp.zeros_like(loss_ref)

    z = z_ref[...]
    zb = z.astype(jnp.bfloat16)
    znorm = jnp.sum(z * z, axis=1, keepdims=True)

    m = z.shape[0]
    acc = jnp.full((m, 1), jnp.inf, jnp.float32)
    idx = jnp.zeros((m, 1), jnp.int32)
    for w in range(k // _W):
        mm = jax.lax.dot_general(
            zb, cbh[w * _W:(w + 1) * _W, :], (((1,), (1,)), ((), ())),
            preferred_element_type=jnp.float32)
        dist = znorm - 2.0 * mm + cnorm_ref[0, w * _W:(w + 1) * _W][None, :]
        wmin = jnp.min(dist, axis=1, keepdims=True)
        col = jax.lax.broadcasted_iota(jnp.int32, dist.shape, 1)
        warg = jnp.min(jnp.where(dist == wmin, col, _W), axis=1,
                       keepdims=True) + w * _W
        upd = wmin < acc
        acc = jnp.where(upd, wmin.astype(jnp.bfloat16).astype(jnp.float32), acc)
        idx = jnp.where(upd, warg, idx)

    ids = idx[:, 0]
    ids_ref[...] = ids.reshape(1, 1, -1)

    col = jax.lax.broadcasted_iota(jnp.int32, (m, k), 1)
    onehot = (col == ids[:, None]).astype(jnp.float32)
    zq = jax.lax.dot_general(
        onehot, cb, (((1,), (0,)), ((), ())),
        preferred_element_type=jnp.float32,
        precision=jax.lax.Precision.HIGHEST)
    zq_ref[...] = zq

    d = zq - z
    loss_ref[...] += jnp.sum(d * d).reshape(1, 1)


def kernel(z, codebook):
    B, S, D = z.shape
    K = codebook.shape[0]
    N = B * S
    z_flat = z.reshape(N, D)
    nblk = N // _M

    zq_flat, ids_blk, loss_sum = pl.pallas_call(
        _vq_kernel,
        grid=(nblk,),
        in_specs=[
            pl.BlockSpec((_M, D), lambda i: (i, 0)),
            pl.BlockSpec((K, D), lambda i: (0, 0)),
        ],
        out_specs=[
            pl.BlockSpec((_M, D), lambda i: (i, 0)),
            pl.BlockSpec((1, 1, _M), lambda i: (i, 0, 0)),
            pl.BlockSpec((1, 1), lambda i: (0, 0)),
        ],
        out_shape=[
            jax.ShapeDtypeStruct((N, D), jnp.float32),
            jax.ShapeDtypeStruct((nblk, 1, _M), jnp.int32),
            jax.ShapeDtypeStruct((1, 1), jnp.float32),
        ],
        scratch_shapes=[pltpu.VMEM((1, K), jnp.float32)],
    )(z_flat, codebook)

    z_q_out = zq_flat.reshape(B, S, D)
    ids = ids_blk.reshape(B, S)
    loss = (loss_sum[0, 0] * (_COMMITMENT_COST / (N * D))).reshape(())
    return (z_q_out, ids, loss)


# speedup vs baseline: 1.8010x; 1.8010x over previous
import jax
import jax.numpy as jnp
from jax.experimental import pallas as pl
from jax.experimental.pallas import tpu as pltpu

_COMMITMENT_COST = 0.25
_M = 256
_W = 2048


def _exact_sq_norm_t(xt):
    s = jnp.zeros_like(xt[0:1, :])
    c = jnp.zeros_like(s)
    for j in range(xt.shape[0]):
        y = xt[j:j + 1, :] * xt[j:j + 1, :]
        t = s + y
        bb = t - s
        err = (s - (t - bb)) + (y - bb)
        c = c + err
        s = t
    return s + c


def _vq_kernel(z_ref, zt_ref, cb_ref, cbt_ref, zq_ref, ids_ref, loss_ref,
               cnorm_ref):
    i = pl.program_id(0)
    cb = cb_ref[...]
    k = cb.shape[0]
    cbh = cb.astype(jnp.bfloat16)

    @pl.when(i == 0)
    def _init():
        cnorm_ref[...] = _exact_sq_norm_t(cbt_ref[...])
        loss_ref[...] = jnp.zeros_like(loss_ref)

    z = z_ref[...]
    zb = z.astype(jnp.bfloat16)
    znorm = _exact_sq_norm_t(zt_ref[...]).reshape(-1, 1)

    m = z.shape[0]
    acc = jnp.full((m, 1), jnp.inf, jnp.float32)
    idx = jnp.zeros((m, 1), jnp.int32)
    for w in range(k // _W):
        mm = jax.lax.dot_general(
            zb, cbh[w * _W:(w + 1) * _W, :], (((1,), (1,)), ((), ())),
            preferred_element_type=jnp.float32)
        dist = znorm - 2.0 * mm + cnorm_ref[0, w * _W:(w + 1) * _W][None, :]
        wmin = jnp.min(dist, axis=1, keepdims=True)
        col = jax.lax.broadcasted_iota(jnp.int32, dist.shape, 1)
        warg = jnp.min(jnp.where(dist == wmin, col, _W), axis=1,
                       keepdims=True) + w * _W
        upd = wmin < acc
        acc = jnp.where(upd, wmin.astype(jnp.bfloat16).astype(jnp.float32), acc)
        idx = jnp.where(upd, warg, idx)

    ids = idx[:, 0]
    ids_ref[...] = ids.reshape(1, 1, -1)

    col = jax.lax.broadcasted_iota(jnp.int32, (m, k), 1)
    onehot = (col == ids[:, None]).astype(jnp.bfloat16)
    cblo = (cb - cbh.astype(jnp.float32)).astype(jnp.bfloat16)
    dn = (((1,), (0,)), ((), ()))
    zq = (jax.lax.dot_general(onehot, cbh, dn, preferred_element_type=jnp.float32)
          + jax.lax.dot_general(onehot, cblo, dn, preferred_element_type=jnp.float32))
    zq_ref[...] = zq

    d = zq - z
    loss_ref[...] += jnp.sum(d * d).reshape(1, 1)


def kernel(z, codebook):
    B, S, D = z.shape
    K = codebook.shape[0]
    N = B * S
    z_flat = z.reshape(N, D)
    nblk = N // _M

    zq_flat, ids_blk, loss_sum = pl.pallas_call(
        _vq_kernel,
        grid=(nblk,),
        in_specs=[
            pl.BlockSpec((_M, D), lambda i: (i, 0)),
            pl.BlockSpec((D, _M), lambda i: (0, i)),
            pl.BlockSpec((K, D), lambda i: (0, 0)),
            pl.BlockSpec((D, K), lambda i: (0, 0)),
        ],
        out_specs=[
            pl.BlockSpec((_M, D), lambda i: (i, 0)),
            pl.BlockSpec((1, 1, _M), lambda i: (i, 0, 0)),
            pl.BlockSpec((1, 1), lambda i: (0, 0)),
        ],
        out_shape=[
            jax.ShapeDtypeStruct((N, D), jnp.float32),
            jax.ShapeDtypeStruct((nblk, 1, _M), jnp.int32),
            jax.ShapeDtypeStruct((1, 1), jnp.float32),
        ],
        scratch_shapes=[pltpu.VMEM((1, K), jnp.float32)],
    )(z_flat, z_flat.T, codebook, codebook.T)

    z_q_out = zq_flat.reshape(B, S, D)
    ids = ids_blk.reshape(B, S)
    loss = (loss_sum[0, 0] * (_COMMITMENT_COST / (N * D))).reshape(())
    return (z_q_out, ids, loss)


# speedup vs baseline: 1.8561x; 1.0306x over previous
import jax
import jax.numpy as jnp
from jax.experimental import pallas as pl
from jax.experimental.pallas import tpu as pltpu

_COMMITMENT_COST = 0.25
_M = 256
_W = 2048


def _exact_sq_norm_t(xt):
    s = jnp.zeros_like(xt[0:1, :])
    c = jnp.zeros_like(s)
    for j in range(xt.shape[0]):
        y = xt[j:j + 1, :] * xt[j:j + 1, :]
        t = s + y
        bb = t - s
        err = (s - (t - bb)) + (y - bb)
        c = c + err
        s = t
    return s + c


def _cnorm_kernel(cbt_ref, cn_ref):
    cn_ref[...] = _exact_sq_norm_t(cbt_ref[...])


def _vq_kernel(z_ref, zt_ref, cb_ref, cn_ref, zq_ref, ids_ref, loss_ref):
    cb = cb_ref[...]
    k = cb.shape[0]
    cbh = cb.astype(jnp.bfloat16)

    z = z_ref[...]
    zb = z.astype(jnp.bfloat16)
    znorm = _exact_sq_norm_t(zt_ref[...]).reshape(-1, 1)

    m = z.shape[0]
    acc = jnp.full((m, 1), jnp.inf, jnp.float32)
    idx = jnp.zeros((m, 1), jnp.int32)
    for w in range(k // _W):
        mm = jax.lax.dot_general(
            zb, cbh[w * _W:(w + 1) * _W, :], (((1,), (1,)), ((), ())),
            preferred_element_type=jnp.float32)
        dist = znorm - 2.0 * mm + cn_ref[0, w * _W:(w + 1) * _W][None, :]
        wmin = jnp.min(dist, axis=1, keepdims=True)
        col = jax.lax.broadcasted_iota(jnp.int32, dist.shape, 1)
        warg = jnp.min(jnp.where(dist == wmin, col, _W), axis=1,
                       keepdims=True) + w * _W
        upd = wmin < acc
        acc = jnp.where(upd, wmin.astype(jnp.bfloat16).astype(jnp.float32), acc)
        idx = jnp.where(upd, warg, idx)

    ids = idx[:, 0]
    ids_ref[...] = ids.reshape(1, 1, -1)

    col = jax.lax.broadcasted_iota(jnp.int32, (m, k), 1)
    onehot = (col == ids[:, None]).astype(jnp.bfloat16)
    cblo = (cb - cbh.astype(jnp.float32)).astype(jnp.bfloat16)
    dn = (((1,), (0,)), ((), ()))
    zq = (jax.lax.dot_general(onehot, cbh, dn, preferred_element_type=jnp.float32)
          + jax.lax.dot_general(onehot, cblo, dn, preferred_element_type=jnp.float32))
    zq_ref[...] = zq

    d = zq - z
    loss_ref[...] = jnp.sum(d * d).reshape(1, 1, 1)


def kernel(z, codebook):
    B, S, D = z.shape
    K = codebook.shape[0]
    N = B * S
    z_flat = z.reshape(N, D)
    nblk = N // _M

    cnorm = pl.pallas_call(
        _cnorm_kernel,
        out_shape=jax.ShapeDtypeStruct((1, K), jnp.float32),
    )(codebook.T)

    zq_flat, ids_blk, loss_part = pl.pallas_call(
        _vq_kernel,
        grid=(nblk,),
        in_specs=[
            pl.BlockSpec((_M, D), lambda i: (i, 0)),
            pl.BlockSpec((D, _M), lambda i: (0, i)),
            pl.BlockSpec((K, D), lambda i: (0, 0)),
            pl.BlockSpec((1, K), lambda i: (0, 0)),
        ],
        out_specs=[
            pl.BlockSpec((_M, D), lambda i: (i, 0)),
            pl.BlockSpec((1, 1, _M), lambda i: (i, 0, 0)),
            pl.BlockSpec((1, 1, 1), lambda i: (i, 0, 0)),
        ],
        out_shape=[
            jax.ShapeDtypeStruct((N, D), jnp.float32),
            jax.ShapeDtypeStruct((nblk, 1, _M), jnp.int32),
            jax.ShapeDtypeStruct((nblk, 1, 1), jnp.float32),
        ],
        compiler_params=pltpu.CompilerParams(
            dimension_semantics=("parallel",)),
    )(z_flat, z_flat.T, codebook, cnorm)

    z_q_out = zq_flat.reshape(B, S, D)
    ids = ids_blk.reshape(B, S)
    loss = (jnp.sum(loss_part) * (_COMMITMENT_COST / (N * D))).reshape(())
    return (z_q_out, ids, loss)
